# baseline (speedup 1.0000x reference)
import functools

import jax
import jax.numpy as jnp
from jax import lax
from jax.experimental import pallas as pl
from jax.experimental.pallas import tpu as pltpu
from jax.experimental.pallas import tpu_sc as plsc

EMBED_DIM = 64
NUM_WORKERS = 32


def _build_gather(batch: int, hist: int):
    rows_per_w = batch // NUM_WORKERS
    mesh = plsc.VectorSubcoreMesh(core_axis_name="c", subcore_axis_name="s")

    @functools.partial(
        pl.kernel,
        mesh=mesh,
        out_type=jax.ShapeDtypeStruct((batch, hist, EMBED_DIM), jnp.float32),
        scratch_types=[
            pltpu.VMEM((rows_per_w, hist), jnp.int32),
            pltpu.VMEM((hist, EMBED_DIM), jnp.float32),
            pltpu.VMEM((hist, EMBED_DIM), jnp.float32),
            pltpu.SemaphoreType.DMA,
            pltpu.SemaphoreType.DMA,
            pltpu.SemaphoreType.DMA,
            pltpu.SemaphoreType.DMA,
        ],
        compiler_params=pltpu.CompilerParams(use_tc_tiling_on_sc=False),
    )
    def gather_kernel(seq_hbm, table_hbm, out_hbm, idx_v, rows0, rows1,
                      sg0, sg1, so0, so1):
        wid = lax.axis_index("s") * 2 + lax.axis_index("c")
        base = wid * rows_per_w
        pltpu.sync_copy(seq_hbm.at[pl.ds(base, rows_per_w)], idx_v)

        def gather_desc(r, rows, sem):
            return pltpu.make_async_copy(table_hbm.at[idx_v.at[r]], rows, sem)

        def out_desc(r, rows, sem):
            return pltpu.make_async_copy(rows, out_hbm.at[base + r], sem)

        gather_desc(0, rows0, sg0).start()

        def body(g, carry):
            for b, rows, sg, so in ((0, rows0, sg0, so0), (1, rows1, sg1, so1)):
                r = 2 * g + b
                rows_o, sg_o, so_o = (rows1, sg1, so1) if b == 0 else (rows0, sg0, so0)
                gather_desc(r, rows, sg).wait()
                out_desc(r, rows, so).start()
                @pl.when(r >= 1)
                def _():
                    out_desc(r - 1, rows_o, so_o).wait()
                @pl.when(r < rows_per_w - 1)
                def _():
                    gather_desc(r + 1, rows_o, sg_o).start()
            return carry

        lax.fori_loop(0, rows_per_w // 2, body, 0)
        out_desc(rows_per_w - 1, rows1, so1).wait()

    return gather_kernel


def kernel(seq, node2vec):
    batch, hist = seq.shape
    return _build_gather(batch, hist)(seq.astype(jnp.int32), node2vec)
